# baseline (speedup 1.0000x reference)
import functools

import jax
import jax.numpy as jnp
from jax import lax
from jax.experimental import pallas as pl
from jax.experimental.pallas import tpu as pltpu
from jax.experimental.pallas import tpu_sc as plsc

HIDDEN = 2048
NUM_WORKERS = 32
CHUNK = 8
NBUF = 4


def _emb_body(bpw, nchunk, wpr, idx_hbm, tab_hbm, out_hbm, idx_v, rows_v, *sems):
    gsems = sems[:NBUF]
    ssems = sems[NBUF:]
    wid = lax.axis_index("s") * 2 + lax.axis_index("c")
    base = wid * bpw
    pltpu.sync_copy(
        idx_hbm.at[wid // wpr, pl.ds((wid % wpr) * bpw, bpw)], idx_v)

    def gather(c, b):
        return pltpu.async_copy(
            tab_hbm.at[idx_v.at[pl.ds(c * CHUNK, CHUNK)]],
            rows_v.at[b], gsems[b])

    def store(c, b):
        return pltpu.async_copy(
            rows_v.at[b],
            out_hbm.at[pl.ds(base + c * CHUNK, CHUNK)], ssems[b])

    gcopy = [gather(b, b) for b in range(NBUF)]
    scopy = [None] * NBUF
    for c in range(nchunk):
        b = c % NBUF
        gcopy[b].wait()
        scopy[b] = store(c, b)
        if c + NBUF < nchunk:
            scopy[b].wait()
            gcopy[b] = gather(c + NBUF, b)
    for c in range(nchunk - NBUF, nchunk):
        scopy[c % NBUF].wait()


def kernel(input_ids, wte):
    batch, seq = input_ids.shape
    b_total = batch * seq
    idx = input_ids.astype(jnp.int32)
    bpw = b_total // NUM_WORKERS
    nchunk = bpw // CHUNK
    wpr = seq // bpw

    mesh = plsc.VectorSubcoreMesh(core_axis_name="c", subcore_axis_name="s")
    emb = functools.partial(
        pl.kernel,
        mesh=mesh,
        out_type=jax.ShapeDtypeStruct((b_total, HIDDEN), jnp.float32),
        scratch_types=[
            pltpu.VMEM((bpw,), jnp.int32),
            pltpu.VMEM((NBUF, CHUNK, HIDDEN), jnp.float32),
        ] + [pltpu.SemaphoreType.DMA] * (2 * NBUF),
    )(functools.partial(_emb_body, bpw, nchunk, wpr))

    out = emb(idx, wte)
    return out.reshape(batch, seq, HIDDEN)


# speedup vs baseline: 1.0090x; 1.0090x over previous
import functools

import jax
import jax.numpy as jnp
from jax import lax
from jax.experimental import pallas as pl
from jax.experimental.pallas import tpu as pltpu
from jax.experimental.pallas import tpu_sc as plsc

HIDDEN = 2048
NUM_WORKERS = 32
CHUNK = 8
NBUF = 7


def _emb_body(bpw, nchunk, wpr, idx_hbm, tab_hbm, out_hbm, idx_v, rows_v, *sems):
    gsems = sems[:NBUF]
    ssems = sems[NBUF:]
    wid = lax.axis_index("s") * 2 + lax.axis_index("c")
    base = wid * bpw
    pltpu.sync_copy(
        idx_hbm.at[wid // wpr, pl.ds((wid % wpr) * bpw, bpw)], idx_v)

    def gather(c, b):
        return pltpu.async_copy(
            tab_hbm.at[idx_v.at[pl.ds(c * CHUNK, CHUNK)]],
            rows_v.at[b], gsems[b])

    def store(c, b):
        return pltpu.async_copy(
            rows_v.at[b],
            out_hbm.at[pl.ds(base + c * CHUNK, CHUNK)], ssems[b])

    gcopy = [gather(b, b) for b in range(NBUF)]
    scopy = [None] * NBUF
    for c in range(nchunk):
        b = c % NBUF
        gcopy[b].wait()
        scopy[b] = store(c, b)
        if c + NBUF < nchunk:
            scopy[b].wait()
            gcopy[b] = gather(c + NBUF, b)
    for c in range(nchunk - NBUF, nchunk):
        scopy[c % NBUF].wait()


def kernel(input_ids, wte):
    batch, seq = input_ids.shape
    b_total = batch * seq
    idx = input_ids.astype(jnp.int32)
    bpw = b_total // NUM_WORKERS
    nchunk = bpw // CHUNK
    wpr = seq // bpw

    mesh = plsc.VectorSubcoreMesh(core_axis_name="c", subcore_axis_name="s")
    emb = functools.partial(
        pl.kernel,
        mesh=mesh,
        out_type=jax.ShapeDtypeStruct((b_total, HIDDEN), jnp.float32),
        scratch_types=[
            pltpu.VMEM((bpw,), jnp.int32),
            pltpu.VMEM((NBUF, CHUNK, HIDDEN), jnp.float32),
        ] + [pltpu.SemaphoreType.DMA] * (2 * NBUF),
    )(functools.partial(_emb_body, bpw, nchunk, wpr))

    out = emb(idx, wte)
    return out.reshape(batch, seq, HIDDEN)


# speedup vs baseline: 1.0125x; 1.0034x over previous
import functools

import jax
import jax.numpy as jnp
from jax import lax
from jax.experimental import pallas as pl
from jax.experimental.pallas import tpu as pltpu
from jax.experimental.pallas import tpu_sc as plsc

HIDDEN = 2048
NUM_WORKERS = 32
CHUNK = 8
NBUF = 6


def _emb_body(bpw, nchunk, idx_hbm, tab_hbm, out_hbm, idx_v, rows_v, *sems):
    gsems = sems[:NBUF]
    ssems = sems[NBUF:]
    wid = lax.axis_index("s") * 2 + lax.axis_index("c")
    base = wid * bpw
    head = NBUF * CHUNK
    head_cp = pltpu.async_copy(
        idx_hbm.at[pl.ds(base, head)], idx_v.at[pl.ds(0, head)],
        sems[2 * NBUF])
    tail_cp = pltpu.async_copy(
        idx_hbm.at[pl.ds(base + head, bpw - head)],
        idx_v.at[pl.ds(head, bpw - head)], sems[2 * NBUF + 1])
    head_cp.wait()

    def gather(c, b):
        return pltpu.async_copy(
            tab_hbm.at[idx_v.at[pl.ds(c * CHUNK, CHUNK)]],
            rows_v.at[b], gsems[b])

    def store(c, b):
        return pltpu.async_copy(
            rows_v.at[b],
            out_hbm.at[pl.ds(base + c * CHUNK, CHUNK)], ssems[b])

    gcopy = [gather(b, b) for b in range(NBUF)]
    tail_cp.wait()
    scopy = [None] * NBUF
    for c in range(nchunk):
        b = c % NBUF
        gcopy[b].wait()
        scopy[b] = store(c, b)
        if c + NBUF < nchunk:
            scopy[b].wait()
            gcopy[b] = gather(c + NBUF, b)
    for c in range(nchunk - NBUF, nchunk):
        scopy[c % NBUF].wait()


def kernel(input_ids, wte):
    batch, seq = input_ids.shape
    b_total = batch * seq
    idx = input_ids.reshape(b_total).astype(jnp.int32)
    bpw = b_total // NUM_WORKERS
    nchunk = bpw // CHUNK

    mesh = plsc.VectorSubcoreMesh(core_axis_name="c", subcore_axis_name="s")
    emb = functools.partial(
        pl.kernel,
        mesh=mesh,
        out_type=jax.ShapeDtypeStruct((b_total, HIDDEN), jnp.float32),
        scratch_types=[
            pltpu.VMEM((bpw,), jnp.int32),
            pltpu.VMEM((NBUF, CHUNK, HIDDEN), jnp.float32),
        ] + [pltpu.SemaphoreType.DMA] * (2 * NBUF + 2),
    )(functools.partial(_emb_body, bpw, nchunk))

    out = emb(idx, wte)
    return out.reshape(batch, seq, HIDDEN)
